# baseline (speedup 1.0000x reference)
import functools

import jax
import jax.numpy as jnp
from jax import lax
from jax.experimental import pallas as pl
from jax.experimental.pallas import tpu as pltpu
from jax.experimental.pallas import tpu_sc as plsc

N = 50000
D = 64
E = 800000
NL = 100000
NC, NS = 2, 16

NHALF = 25600
NPAD = 2 * NHALF
TR = NHALF // NS
EPT = E // NS
CB = 80
NT = NC * NS
BKT = NPAD // NT
EBP = 2000
NBP = E // EBP
FB = 2048
LCAP = 802816
SB = 2048
ECL = 128
NLP = 102400
LPT = NLP // (NC * NS)
LCB = 128
LNCH = LPT // LCB

_MESH = plsc.VectorSubcoreMesh(
    core_axis_name="c", subcore_axis_name="s", num_cores=NC, num_subcores=NS
)

_f32 = jnp.float32
_i32 = jnp.int32

_CP = pltpu.CompilerParams(
    use_tc_tiling_on_sc=False, needs_layout_passes=False
)


def _rsqrt16(x):
    s = jnp.maximum(x, 1.0)
    for _ in range(18):
        s = 0.5 * (s + x / s)
    return jnp.where(x > 0.0, 1.0 / s, 0.0)


@functools.partial(
    pl.kernel,
    out_type=(
        jax.ShapeDtypeStruct((NPAD,), _f32),
        jax.ShapeDtypeStruct((NPAD, D), _f32),
        jax.ShapeDtypeStruct((NPAD, D), _f32),
    ),
    mesh=_MESH,
    compiler_params=_CP,
    scratch_types=(
        pltpu.VMEM_SHARED((NPAD,), _f32),
        pltpu.VMEM((NPAD // NS,), _f32),
        pltpu.VMEM((CB,), _f32),
        pltpu.VMEM((10000,), _i32),
        pltpu.VMEM((NPAD // NS,), _f32),
        pltpu.VMEM((CB,), _f32),
        pltpu.VMEM((CB, D), _f32),
        pltpu.VMEM((CB, D), _f32),
        pltpu.VMEM((CB, D), _f32),
        pltpu.VMEM((16,), _f32),
    ),
)
def _k0(col_hbm, emb_hbm, w_hbm, dinv_o, xt0_o, out0_o,
        deg_sh, zb, ones, colv, degv, dv, embv, xtv, ov, wv):
    c = lax.axis_index("c")
    s = lax.axis_index("s")
    spt = NPAD // NS

    def zb_body(i, _):
        zb[pl.ds(16 * i, 16)] = jnp.zeros((16,), _f32)
        return _
    lax.fori_loop(0, spt // 16, zb_body, None)
    pltpu.sync_copy(zb.at[pl.ds(0, spt)], deg_sh.at[pl.ds(s * spt, spt)])
    for k in range(CB // 16):
        ones[pl.ds(16 * k, 16)] = jnp.ones((16,), _f32)
    pltpu.sync_copy(w_hbm, wv)
    plsc.subcore_barrier()

    for b in range(5):
        pltpu.sync_copy(col_hbm.at[pl.ds(s * EPT + b * 10000, 10000)], colv)

        def deg_body(j, _):
            pltpu.sync_copy(
                ones, deg_sh.at[colv.at[pl.ds(j * CB, CB)]], add=True
            )
            return _
        lax.fori_loop(0, 10000 // CB, deg_body, None)
    plsc.subcore_barrier()

    @pl.when(c == 0)
    def _():
        pltpu.sync_copy(deg_sh.at[pl.ds(s * spt, spt)], degv)

        def nwt(i, _):
            degv[pl.ds(16 * i, 16)] = _rsqrt16(degv[pl.ds(16 * i, 16)])
            return _
        lax.fori_loop(0, spt // 16, nwt, None)
        pltpu.sync_copy(degv, dinv_o.at[pl.ds(s * spt, spt)])

    g0 = c * NHALF + s * TR

    def x_body(ch, _):
        gb = g0 + ch * CB
        eb = jnp.minimum(gb, N - CB)
        pltpu.sync_copy(emb_hbm.at[pl.ds(eb, CB)], embv)
        pltpu.sync_copy(deg_sh.at[pl.ds(gb, CB)], dv)
        for k in range(CB // 16):
            dv[pl.ds(16 * k, 16)] = _rsqrt16(dv[pl.ds(16 * k, 16)])
        w0 = wv[pl.ds(0, 16)][0]

        def row_grp(g, _):
            dvec = dv[pl.ds(16 * g, 16)]
            for k in range(16):
                d = dvec[k]
                r = 16 * g + k
                for v in range(D // 16):
                    e = embv[r, pl.ds(16 * v, 16)]
                    xtv[r, pl.ds(16 * v, 16)] = d * e
                    ov[r, pl.ds(16 * v, 16)] = w0 * e
            return _
        lax.fori_loop(0, CB // 16, row_grp, None)
        pltpu.sync_copy(xtv, xt0_o.at[pl.ds(gb, CB)])
        pltpu.sync_copy(ov, out0_o.at[pl.ds(gb, CB)])
        return _
    lax.fori_loop(0, TR // CB, x_body, None)


RCAP = 4096


@functools.partial(
    pl.kernel,
    out_type=(
        jax.ShapeDtypeStruct((NT, LCAP), _i32),
        jax.ShapeDtypeStruct((NT, LCAP), _i32),
        jax.ShapeDtypeStruct((NT, 16), _i32),
    ),
    mesh=_MESH,
    compiler_params=_CP,
    scratch_types=(
        pltpu.VMEM((EBP,), _i32),
        pltpu.VMEM((EBP,), _i32),
        pltpu.VMEM((RCAP,), _i32),
        pltpu.VMEM((RCAP,), _i32),
        pltpu.VMEM((16,), _i32),
    ),
)
def _kpart(row_hbm, col_hbm, rowP_o, clP_o, cnt_o, rowB, clB, rbuf, cbuf,
           cntv):
    c = lax.axis_index("c")
    s = lax.axis_index("s")
    wid = c * NS + s
    base = wid * BKT
    lane = lax.iota(_i32, 16)

    def flush_chunk(wptr):
        wp = pl.multiple_of(wptr, FB)
        pltpu.sync_copy(rbuf.at[pl.ds(0, FB)],
                        rowP_o.at[wid, pl.ds(wp, FB)])
        pltpu.sync_copy(cbuf.at[pl.ds(0, FB)],
                        clP_o.at[wid, pl.ds(wp, FB)])

    def blk(b, carry):
        off, wptr = carry
        e0 = b * EBP
        pltpu.sync_copy(row_hbm.at[pl.ds(e0, EBP)], rowB)
        pltpu.sync_copy(col_hbm.at[pl.ds(e0, EBP)], clB)

        def vec(i, off):
            rv = rowB[pl.ds(16 * i, 16)]
            cv = clB[pl.ds(16 * i, 16)]
            lc = cv - base
            m = (lc >= 0) & (lc < BKT)
            plsc.store_compressed(cbuf.at[pl.ds(off, 16)], lc, mask=m)
            plsc.store_compressed(rbuf.at[pl.ds(off, 16)], rv, mask=m)
            return off + plsc.all_reduce_population_count(m)[0]
        off = lax.fori_loop(0, EBP // 16, vec, off)

        nf = off // FB

        @pl.when(nf > 0)
        def _():
            flush_chunk(wptr)

            def cd(k, _):
                rbuf[pl.ds(16 * k, 16)] = rbuf[pl.ds(FB + 16 * k, 16)]
                cbuf[pl.ds(16 * k, 16)] = cbuf[pl.ds(FB + 16 * k, 16)]
                return _
            lax.fori_loop(0, (off - FB + 15) // 16, cd, None)
        return (off - nf * FB, wptr + nf * FB)

    off, wptr = lax.fori_loop(0, NBP, blk, (jnp.int32(0), jnp.int32(0)))

    pad = (FB - off % FB) % FB
    negv = jnp.full((16,), -1, _i32)
    zv = jnp.zeros((16,), _i32)

    def padb(p, _):
        cbuf[pl.ds(off + 16 * p, 16)] = negv
        rbuf[pl.ds(off + 16 * p, 16)] = zv
        return _
    lax.fori_loop(0, (pad + 15) // 16, padb, None)
    off = off + pad
    nf = off // FB

    @pl.when(nf > 0)
    def _():
        flush_chunk(wptr)
    wptr = wptr + nf * FB
    cntv[pl.ds(0, 16)] = jnp.where(lane == 0, wptr, 0)
    pltpu.sync_copy(cntv, cnt_o.at[wid])


def _make_layer(widx: int, last: bool):
    outs = [jax.ShapeDtypeStruct((NPAD, D), _f32)]
    if not last:
        outs.append(jax.ShapeDtypeStruct((NPAD, D), _f32))

    @functools.partial(
        pl.kernel,
        out_type=tuple(outs),
        mesh=_MESH,
        compiler_params=_CP,
        scratch_types=(
            pltpu.VMEM((BKT, D), _f32),
            pltpu.VMEM((SB,), _i32),
            pltpu.VMEM((SB,), _i32),
            pltpu.VMEM((ECL, D), _f32),
            pltpu.VMEM((ECL, D), _f32),
            pltpu.VMEM((CB,), _f32),
            pltpu.VMEM((16,), _f32),
            pltpu.VMEM((16,), _i32),
            pltpu.SemaphoreType.DMA,
            pltpu.SemaphoreType.DMA,
        ),
    )
    def _layer(rowP, clP, cnt_hbm, xt_in, out_in, dinv_hbm, w_hbm,
               out_o, *rest):
        if last:
            (acc, rowL, clL, msgA, msgB, dv, wv, cntv, gsA, gsB) = rest
            xt_o = None
        else:
            (xt_o, acc, rowL, clL, msgA, msgB, dv, wv, cntv,
             gsA, gsB) = rest
        c = lax.axis_index("c")
        s = lax.axis_index("s")
        wid = c * NS + s

        z = jnp.zeros((16,), _f32)

        def z_body(i, _):
            for v in range(D // 16):
                acc[i, pl.ds(16 * v, 16)] = z
            return _
        lax.fori_loop(0, BKT, z_body, None)
        pltpu.sync_copy(w_hbm, wv)
        pltpu.sync_copy(cnt_hbm.at[wid], cntv)
        nblk = cntv[pl.ds(0, 16)][0] // SB

        def fire_g(j, buf, sem):
            pltpu.async_copy(xt_in.at[rowL.at[pl.ds(j * ECL, ECL)]],
                             buf, sem)

        def drain_g(j, buf, sem):
            pltpu.make_async_copy(xt_in.at[rowL.at[pl.ds(j * ECL, ECL)]],
                                  buf, sem).wait()

        lane = lax.iota(_i32, 16)

        def add_chunk(j, buf):
            def g_body(g, _):
                lcv = clL[pl.ds(j * ECL + 16 * g, 16)]
                m = lcv >= 0
                ev = lane + 16 * g
                for d in range(D):
                    dd = jnp.full((16,), d, _i32)
                    x = plsc.load_gather(buf, [ev, dd])
                    plsc.addupdate_scatter(acc, [lcv, dd], x, mask=m)
                return _
            lax.fori_loop(0, ECL // 16, g_body, None)

        def blk_body(b, _):
            e0 = b * SB
            pltpu.sync_copy(rowP.at[wid, pl.ds(e0, SB)], rowL)
            pltpu.sync_copy(clP.at[wid, pl.ds(e0, SB)], clL)
            fire_g(0, msgA, gsA)
            fire_g(1, msgB, gsB)

            def pair_body(i, _):
                j = 2 * i
                drain_g(j, msgA, gsA)
                add_chunk(j, msgA)
                fire_g(j + 2, msgA, gsA)
                drain_g(j + 1, msgB, gsB)
                add_chunk(j + 1, msgB)
                fire_g(j + 3, msgB, gsB)
                return _
            lax.fori_loop(0, SB // ECL // 2 - 1, pair_body, None)
            j = SB // ECL - 2
            drain_g(j, msgA, gsA)
            add_chunk(j, msgA)
            drain_g(j + 1, msgB, gsB)
            add_chunk(j + 1, msgB)
            return _
        lax.fori_loop(0, nblk, blk_body, None)

        g0 = wid * BKT
        w = wv[pl.ds(0, 16)][widx]

        def wb_body(ch, _):
            gb = g0 + ch * CB
            pltpu.sync_copy(dinv_hbm.at[pl.ds(gb, CB)], dv)
            pltpu.sync_copy(out_in.at[pl.ds(gb, CB)], msgA.at[pl.ds(0, CB)])

            def row_grp(g, _):
                dvec = dv[pl.ds(16 * g, 16)]
                for k in range(16):
                    d = dvec[k]
                    r = 16 * g + k
                    for v in range(D // 16):
                        a = acc[ch * CB + r, pl.ds(16 * v, 16)]
                        xn = d * a
                        msgA[r, pl.ds(16 * v, 16)] = (
                            msgA[r, pl.ds(16 * v, 16)] + w * xn
                        )
                        if not last:
                            msgB[r, pl.ds(16 * v, 16)] = d * xn
                return _
            lax.fori_loop(0, CB // 16, row_grp, None)
            pltpu.sync_copy(msgA.at[pl.ds(0, CB)], out_o.at[pl.ds(gb, CB)])
            if not last:
                pltpu.sync_copy(msgB.at[pl.ds(0, CB)],
                                xt_o.at[pl.ds(gb, CB)])
            return _
        lax.fori_loop(0, BKT // CB, wb_body, None)

    return _layer


@functools.partial(
    pl.kernel,
    out_type=jax.ShapeDtypeStruct((NLP,), _f32),
    mesh=_MESH,
    compiler_params=_CP,
    scratch_types=(
        pltpu.VMEM((LCB,), _i32),
        pltpu.VMEM((LCB,), _i32),
        pltpu.VMEM((LCB, D), _f32),
        pltpu.VMEM((LCB, D), _f32),
        pltpu.VMEM((LCB,), _f32),
    ),
)
def _k4(out_hbm, lsrc, ldst, res_o, siv, div_, av, bv, rv):
    c = lax.axis_index("c")
    s = lax.axis_index("s")
    wid = s * NC + c
    base0 = wid * LPT

    def ch_body(ch, _):
        eb = base0 + ch * LCB
        pltpu.sync_copy(lsrc.at[pl.ds(eb, LCB)], siv)
        pltpu.sync_copy(ldst.at[pl.ds(eb, LCB)], div_)
        pltpu.sync_copy(out_hbm.at[siv], av)
        pltpu.sync_copy(out_hbm.at[div_], bv)

        lane = lax.iota(_i32, 16)

        def row_grp(g, _):
            t = jnp.zeros((16,), _f32)
            for k in range(16):
                r = 16 * g + k
                acc = av[r, pl.ds(0, 16)] * bv[r, pl.ds(0, 16)]
                for v in range(1, D // 16):
                    acc = acc + (av[r, pl.ds(16 * v, 16)]
                                 * bv[r, pl.ds(16 * v, 16)])
                t = jnp.where(lane == k, jnp.sum(acc), t)
            rv[pl.ds(16 * g, 16)] = t
            return _
        lax.fori_loop(0, LCB // 16, row_grp, None)
        pltpu.sync_copy(rv, res_o.at[pl.ds(eb, LCB)])
        return _
    lax.fori_loop(0, LNCH, ch_body, None)


_LAYER1 = _make_layer(1, last=False)
_LAYER2 = _make_layer(2, last=False)
_LAYER3 = _make_layer(3, last=True)


def kernel(edge_index, edge_label_index, embedding, alpha):
    row = edge_index[0]
    col = edge_index[1]
    w = jax.nn.softmax(alpha, axis=-1)
    w16 = jnp.zeros((16,), _f32).at[:4].set(w)
    lsrc = jnp.zeros((NLP,), _i32).at[:NL].set(edge_label_index[0])
    ldst = jnp.zeros((NLP,), _i32).at[:NL].set(edge_label_index[1])

    dinv, xt0, out0 = _k0(col, embedding, w16)
    rowP, clP, cnts = _kpart(row, col)
    out1, xt1 = _LAYER1(rowP, clP, cnts, xt0, out0, dinv, w16)
    out2, xt2 = _LAYER2(rowP, clP, cnts, xt1, out1, dinv, w16)
    (out3,) = _LAYER3(rowP, clP, cnts, xt2, out2, dinv, w16)
    res = _k4(out3, lsrc, ldst)
    return res[:NL]


# speedup vs baseline: 2.0495x; 2.0495x over previous
import functools

import jax
import jax.numpy as jnp
from jax import lax
from jax.experimental import pallas as pl
from jax.experimental.pallas import tpu as pltpu
from jax.experimental.pallas import tpu_sc as plsc

N = 50000
D = 64
E = 800000
NL = 100000
NC, NS = 2, 16

NHALF = 25600
NPAD = 2 * NHALF
TR = NHALF // NS
EPT = E // NS
CB = 80
NT = NC * NS
BKT = NPAD // NT
EBP = 2000
NBP = E // EBP
FB = 2048
LCAP = 802816
SB = 2048
ECL = 128
NLP = 102400
LPT = NLP // (NC * NS)
LCB = 128
LNCH = LPT // LCB

_MESH = plsc.VectorSubcoreMesh(
    core_axis_name="c", subcore_axis_name="s", num_cores=NC, num_subcores=NS
)

_f32 = jnp.float32
_i32 = jnp.int32

_CP = pltpu.CompilerParams(
    use_tc_tiling_on_sc=False, needs_layout_passes=False
)


def _rsqrt16(x):
    s = jnp.maximum(x, 1.0)
    for _ in range(18):
        s = 0.5 * (s + x / s)
    return jnp.where(x > 0.0, 1.0 / s, 0.0)


@functools.partial(
    pl.kernel,
    out_type=(
        jax.ShapeDtypeStruct((NPAD,), _f32),
        jax.ShapeDtypeStruct((NPAD, D), _f32),
        jax.ShapeDtypeStruct((NPAD, D), _f32),
    ),
    mesh=_MESH,
    compiler_params=_CP,
    scratch_types=(
        pltpu.VMEM_SHARED((NPAD,), _f32),
        pltpu.VMEM((NPAD // NS,), _f32),
        pltpu.VMEM((CB,), _f32),
        pltpu.VMEM((10000,), _i32),
        pltpu.VMEM((NPAD // NS,), _f32),
        pltpu.VMEM((CB,), _f32),
        pltpu.VMEM((CB, D), _f32),
        pltpu.VMEM((CB, D), _f32),
        pltpu.VMEM((CB, D), _f32),
        pltpu.VMEM((16,), _f32),
    ),
)
def _k0(col_hbm, emb_hbm, w_hbm, dinv_o, xt0_o, out0_o,
        deg_sh, zb, ones, colv, degv, dv, embv, xtv, ov, wv):
    c = lax.axis_index("c")
    s = lax.axis_index("s")
    spt = NPAD // NS

    def zb_body(i, _):
        zb[pl.ds(16 * i, 16)] = jnp.zeros((16,), _f32)
        return _
    lax.fori_loop(0, spt // 16, zb_body, None)
    pltpu.sync_copy(zb.at[pl.ds(0, spt)], deg_sh.at[pl.ds(s * spt, spt)])
    for k in range(CB // 16):
        ones[pl.ds(16 * k, 16)] = jnp.ones((16,), _f32)
    pltpu.sync_copy(w_hbm, wv)
    plsc.subcore_barrier()

    for b in range(5):
        pltpu.sync_copy(col_hbm.at[pl.ds(s * EPT + b * 10000, 10000)], colv)

        def deg_body(j, _):
            pltpu.sync_copy(
                ones, deg_sh.at[colv.at[pl.ds(j * CB, CB)]], add=True
            )
            return _
        lax.fori_loop(0, 10000 // CB, deg_body, None)
    plsc.subcore_barrier()

    @pl.when(c == 0)
    def _():
        pltpu.sync_copy(deg_sh.at[pl.ds(s * spt, spt)], degv)

        def nwt(i, _):
            degv[pl.ds(16 * i, 16)] = _rsqrt16(degv[pl.ds(16 * i, 16)])
            return _
        lax.fori_loop(0, spt // 16, nwt, None)
        pltpu.sync_copy(degv, dinv_o.at[pl.ds(s * spt, spt)])

    g0 = c * NHALF + s * TR

    def x_body(ch, _):
        gb = g0 + ch * CB
        eb = jnp.minimum(gb, N - CB)
        pltpu.sync_copy(emb_hbm.at[pl.ds(eb, CB)], embv)
        pltpu.sync_copy(deg_sh.at[pl.ds(gb, CB)], dv)
        for k in range(CB // 16):
            dv[pl.ds(16 * k, 16)] = _rsqrt16(dv[pl.ds(16 * k, 16)])
        w0 = wv[pl.ds(0, 16)][0]

        def row_grp(g, _):
            dvec = dv[pl.ds(16 * g, 16)]
            for k in range(16):
                d = dvec[k]
                r = 16 * g + k
                for v in range(D // 16):
                    e = embv[r, pl.ds(16 * v, 16)]
                    xtv[r, pl.ds(16 * v, 16)] = d * e
                    ov[r, pl.ds(16 * v, 16)] = w0 * e
            return _
        lax.fori_loop(0, CB // 16, row_grp, None)
        pltpu.sync_copy(xtv, xt0_o.at[pl.ds(gb, CB)])
        pltpu.sync_copy(ov, out0_o.at[pl.ds(gb, CB)])
        return _
    lax.fori_loop(0, TR // CB, x_body, None)


RCAP = 4096


@functools.partial(
    pl.kernel,
    out_type=(
        jax.ShapeDtypeStruct((NT, LCAP), _i32),
        jax.ShapeDtypeStruct((NT, LCAP), _i32),
        jax.ShapeDtypeStruct((NT, 16), _i32),
    ),
    mesh=_MESH,
    compiler_params=_CP,
    scratch_types=(
        pltpu.VMEM((EBP,), _i32),
        pltpu.VMEM((EBP,), _i32),
        pltpu.VMEM((RCAP,), _i32),
        pltpu.VMEM((RCAP,), _i32),
        pltpu.VMEM((16,), _i32),
    ),
)
def _kpart(row_hbm, col_hbm, rowP_o, clP_o, cnt_o, rowB, clB, rbuf, cbuf,
           cntv):
    c = lax.axis_index("c")
    s = lax.axis_index("s")
    wid = c * NS + s
    base = wid * BKT
    lane = lax.iota(_i32, 16)

    def flush_chunk(wptr):
        wp = pl.multiple_of(wptr, FB)
        pltpu.sync_copy(rbuf.at[pl.ds(0, FB)],
                        rowP_o.at[wid, pl.ds(wp, FB)])
        pltpu.sync_copy(cbuf.at[pl.ds(0, FB)],
                        clP_o.at[wid, pl.ds(wp, FB)])

    def blk(b, carry):
        off, wptr = carry
        e0 = b * EBP
        pltpu.sync_copy(row_hbm.at[pl.ds(e0, EBP)], rowB)
        pltpu.sync_copy(col_hbm.at[pl.ds(e0, EBP)], clB)

        def vec(i, off):
            rv = rowB[pl.ds(16 * i, 16)]
            cv = clB[pl.ds(16 * i, 16)]
            lc = cv - base
            m = (lc >= 0) & (lc < BKT)
            plsc.store_compressed(cbuf.at[pl.ds(off, 16)], lc, mask=m)
            plsc.store_compressed(rbuf.at[pl.ds(off, 16)], rv, mask=m)
            return off + plsc.all_reduce_population_count(m)[0]
        off = lax.fori_loop(0, EBP // 16, vec, off)

        nf = off // FB

        @pl.when(nf > 0)
        def _():
            flush_chunk(wptr)

            def cd(k, _):
                rbuf[pl.ds(16 * k, 16)] = rbuf[pl.ds(FB + 16 * k, 16)]
                cbuf[pl.ds(16 * k, 16)] = cbuf[pl.ds(FB + 16 * k, 16)]
                return _
            lax.fori_loop(0, (off - FB + 15) // 16, cd, None)
        return (off - nf * FB, wptr + nf * FB)

    off, wptr = lax.fori_loop(0, NBP, blk, (jnp.int32(0), jnp.int32(0)))

    pad = (FB - off % FB) % FB
    negv = jnp.full((16,), -1, _i32)
    zv = jnp.zeros((16,), _i32)

    def padb(p, _):
        cbuf[pl.ds(off + 16 * p, 16)] = negv
        rbuf[pl.ds(off + 16 * p, 16)] = zv
        return _
    lax.fori_loop(0, (pad + 15) // 16, padb, None)
    off = off + pad
    nf = off // FB

    @pl.when(nf > 0)
    def _():
        flush_chunk(wptr)
    wptr = wptr + nf * FB
    cntv[pl.ds(0, 16)] = jnp.where(lane == 0, wptr, 0)
    pltpu.sync_copy(cntv, cnt_o.at[wid])


def _make_layer(widx: int, last: bool):
    outs = [jax.ShapeDtypeStruct((NPAD, D), _f32)]
    if not last:
        outs.append(jax.ShapeDtypeStruct((NPAD, D), _f32))

    @functools.partial(
        pl.kernel,
        out_type=tuple(outs),
        mesh=_MESH,
        compiler_params=_CP,
        scratch_types=(
            pltpu.VMEM((BKT + 16, D), _f32),
            pltpu.VMEM((SB,), _i32),
            pltpu.VMEM((SB,), _i32),
            pltpu.VMEM((ECL, D), _f32),
            pltpu.VMEM((ECL, D), _f32),
            pltpu.VMEM((CB,), _f32),
            pltpu.VMEM((16,), _f32),
            pltpu.VMEM((16,), _i32),
            pltpu.SemaphoreType.DMA,
            pltpu.SemaphoreType.DMA,
        ),
    )
    def _layer(rowP, clP, cnt_hbm, xt_in, out_in, dinv_hbm, w_hbm,
               out_o, *rest):
        if last:
            (acc, rowL, clL, msgA, msgB, dv, wv, cntv, gsA, gsB) = rest
            xt_o = None
        else:
            (xt_o, acc, rowL, clL, msgA, msgB, dv, wv, cntv,
             gsA, gsB) = rest
        c = lax.axis_index("c")
        s = lax.axis_index("s")
        wid = c * NS + s

        z = jnp.zeros((16,), _f32)

        def z_body(i, _):
            for v in range(D // 16):
                acc[i, pl.ds(16 * v, 16)] = z
            return _
        lax.fori_loop(0, BKT, z_body, None)
        pltpu.sync_copy(w_hbm, wv)
        pltpu.sync_copy(cnt_hbm.at[wid], cntv)
        nblk = cntv[pl.ds(0, 16)][0] // SB

        def fire_g(j, buf, sem):
            pltpu.async_copy(xt_in.at[rowL.at[pl.ds(j * ECL, ECL)]],
                             buf, sem)

        def drain_g(j, buf, sem):
            pltpu.make_async_copy(xt_in.at[rowL.at[pl.ds(j * ECL, ECL)]],
                                  buf, sem).wait()

        def add_chunk(j, buf):
            def g_body(g, _):
                lcv = clL[pl.ds(j * ECL + 16 * g, 16)]
                lcv = jnp.where(lcv >= 0, lcv, BKT)
                for k in range(16):
                    lc = lcv[k]
                    rr = 16 * g + k
                    for v in range(D // 16):
                        x = buf[rr, pl.ds(16 * v, 16)]
                        plsc.addupdate(acc.at[lc, pl.ds(16 * v, 16)], x)
                return _
            lax.fori_loop(0, ECL // 16, g_body, None)

        def blk_body(b, _):
            e0 = b * SB
            pltpu.sync_copy(rowP.at[wid, pl.ds(e0, SB)], rowL)
            pltpu.sync_copy(clP.at[wid, pl.ds(e0, SB)], clL)
            fire_g(0, msgA, gsA)
            fire_g(1, msgB, gsB)

            def pair_body(i, _):
                j = 2 * i
                drain_g(j, msgA, gsA)
                add_chunk(j, msgA)
                fire_g(j + 2, msgA, gsA)
                drain_g(j + 1, msgB, gsB)
                add_chunk(j + 1, msgB)
                fire_g(j + 3, msgB, gsB)
                return _
            lax.fori_loop(0, SB // ECL // 2 - 1, pair_body, None)
            j = SB // ECL - 2
            drain_g(j, msgA, gsA)
            add_chunk(j, msgA)
            drain_g(j + 1, msgB, gsB)
            add_chunk(j + 1, msgB)
            return _
        lax.fori_loop(0, nblk, blk_body, None)

        g0 = wid * BKT
        w = wv[pl.ds(0, 16)][widx]

        def wb_body(ch, _):
            gb = g0 + ch * CB
            pltpu.sync_copy(dinv_hbm.at[pl.ds(gb, CB)], dv)
            pltpu.sync_copy(out_in.at[pl.ds(gb, CB)], msgA.at[pl.ds(0, CB)])

            def row_grp(g, _):
                dvec = dv[pl.ds(16 * g, 16)]
                for k in range(16):
                    d = dvec[k]
                    r = 16 * g + k
                    for v in range(D // 16):
                        a = acc[ch * CB + r, pl.ds(16 * v, 16)]
                        xn = d * a
                        msgA[r, pl.ds(16 * v, 16)] = (
                            msgA[r, pl.ds(16 * v, 16)] + w * xn
                        )
                        if not last:
                            msgB[r, pl.ds(16 * v, 16)] = d * xn
                return _
            lax.fori_loop(0, CB // 16, row_grp, None)
            pltpu.sync_copy(msgA.at[pl.ds(0, CB)], out_o.at[pl.ds(gb, CB)])
            if not last:
                pltpu.sync_copy(msgB.at[pl.ds(0, CB)],
                                xt_o.at[pl.ds(gb, CB)])
            return _
        lax.fori_loop(0, BKT // CB, wb_body, None)

    return _layer


@functools.partial(
    pl.kernel,
    out_type=jax.ShapeDtypeStruct((NLP,), _f32),
    mesh=_MESH,
    compiler_params=_CP,
    scratch_types=(
        pltpu.VMEM((LCB,), _i32),
        pltpu.VMEM((LCB,), _i32),
        pltpu.VMEM((LCB, D), _f32),
        pltpu.VMEM((LCB, D), _f32),
        pltpu.VMEM((LCB,), _f32),
    ),
)
def _k4(out_hbm, lsrc, ldst, res_o, siv, div_, av, bv, rv):
    c = lax.axis_index("c")
    s = lax.axis_index("s")
    wid = s * NC + c
    base0 = wid * LPT

    def ch_body(ch, _):
        eb = base0 + ch * LCB
        pltpu.sync_copy(lsrc.at[pl.ds(eb, LCB)], siv)
        pltpu.sync_copy(ldst.at[pl.ds(eb, LCB)], div_)
        pltpu.sync_copy(out_hbm.at[siv], av)
        pltpu.sync_copy(out_hbm.at[div_], bv)

        lane = lax.iota(_i32, 16)

        def row_grp(g, _):
            t = jnp.zeros((16,), _f32)
            for k in range(16):
                r = 16 * g + k
                acc = av[r, pl.ds(0, 16)] * bv[r, pl.ds(0, 16)]
                for v in range(1, D // 16):
                    acc = acc + (av[r, pl.ds(16 * v, 16)]
                                 * bv[r, pl.ds(16 * v, 16)])
                t = jnp.where(lane == k, jnp.sum(acc), t)
            rv[pl.ds(16 * g, 16)] = t
            return _
        lax.fori_loop(0, LCB // 16, row_grp, None)
        pltpu.sync_copy(rv, res_o.at[pl.ds(eb, LCB)])
        return _
    lax.fori_loop(0, LNCH, ch_body, None)


_LAYER1 = _make_layer(1, last=False)
_LAYER2 = _make_layer(2, last=False)
_LAYER3 = _make_layer(3, last=True)


def kernel(edge_index, edge_label_index, embedding, alpha):
    row = edge_index[0]
    col = edge_index[1]
    w = jax.nn.softmax(alpha, axis=-1)
    w16 = jnp.zeros((16,), _f32).at[:4].set(w)
    lsrc = jnp.zeros((NLP,), _i32).at[:NL].set(edge_label_index[0])
    ldst = jnp.zeros((NLP,), _i32).at[:NL].set(edge_label_index[1])

    dinv, xt0, out0 = _k0(col, embedding, w16)
    rowP, clP, cnts = _kpart(row, col)
    out1, xt1 = _LAYER1(rowP, clP, cnts, xt0, out0, dinv, w16)
    out2, xt2 = _LAYER2(rowP, clP, cnts, xt1, out1, dinv, w16)
    (out3,) = _LAYER3(rowP, clP, cnts, xt2, out2, dinv, w16)
    res = _k4(out3, lsrc, ldst)
    return res[:NL]


# speedup vs baseline: 3.6526x; 1.7822x over previous
import functools

import jax
import jax.numpy as jnp
from jax import lax
from jax.experimental import pallas as pl
from jax.experimental.pallas import tpu as pltpu
from jax.experimental.pallas import tpu_sc as plsc

N = 50000
D = 64
E = 800000
NL = 100000
NC, NS = 2, 16

NHALF = 25600
NPAD = 2 * NHALF
TR = NHALF // NS
EPT = E // NS
CB = 80
ECB = 80
EB = 2000
NEB = EPT // EB
NCB = EB // ECB
NLP = 102400
LPT = NLP // (NC * NS)
LCB = 128
LNCH = LPT // LCB

_MESH = plsc.VectorSubcoreMesh(
    core_axis_name="c", subcore_axis_name="s", num_cores=NC, num_subcores=NS
)

_f32 = jnp.float32
_i32 = jnp.int32

_CP = pltpu.CompilerParams(
    use_tc_tiling_on_sc=False, needs_layout_passes=False
)


def _rsqrt16(x):
    s = jnp.maximum(x, 1.0)
    for _ in range(18):
        s = 0.5 * (s + x / s)
    return jnp.where(x > 0.0, 1.0 / s, 0.0)


def _zero_rows(buf, rows):
    z = jnp.zeros((16,), _f32)

    def body(r, _):
        for v in range(D // 16):
            buf[r, pl.ds(16 * v, 16)] = z
        return _

    lax.fori_loop(0, rows, body, None)


@functools.partial(
    pl.kernel,
    out_type=(
        jax.ShapeDtypeStruct((NPAD,), _f32),
        jax.ShapeDtypeStruct((NPAD, D), _f32),
        jax.ShapeDtypeStruct((NPAD, D), _f32),
    ),
    mesh=_MESH,
    compiler_params=_CP,
    scratch_types=(
        pltpu.VMEM_SHARED((NPAD,), _f32),
        pltpu.VMEM((NPAD // NS,), _f32),
        pltpu.VMEM((CB,), _f32),
        pltpu.VMEM((10000,), _i32),
        pltpu.VMEM((NPAD // NS,), _f32),
        pltpu.VMEM((CB,), _f32),
        pltpu.VMEM((CB, D), _f32),
        pltpu.VMEM((CB, D), _f32),
        pltpu.VMEM((CB, D), _f32),
        pltpu.VMEM((16,), _f32),
    ),
)
def _k0(col_hbm, emb_hbm, w_hbm, dinv_o, xt0_o, out0_o,
        deg_sh, zb, ones, colv, degv, dv, embv, xtv, ov, wv):
    c = lax.axis_index("c")
    s = lax.axis_index("s")
    spt = NPAD // NS

    def zb_body(i, _):
        zb[pl.ds(16 * i, 16)] = jnp.zeros((16,), _f32)
        return _
    lax.fori_loop(0, spt // 16, zb_body, None)
    pltpu.sync_copy(zb.at[pl.ds(0, spt)], deg_sh.at[pl.ds(s * spt, spt)])
    for k in range(CB // 16):
        ones[pl.ds(16 * k, 16)] = jnp.ones((16,), _f32)
    pltpu.sync_copy(w_hbm, wv)
    plsc.subcore_barrier()

    for b in range(5):
        pltpu.sync_copy(col_hbm.at[pl.ds(s * EPT + b * 10000, 10000)], colv)

        def deg_body(j, _):
            pltpu.sync_copy(
                ones, deg_sh.at[colv.at[pl.ds(j * CB, CB)]], add=True
            )
            return _
        lax.fori_loop(0, 10000 // CB, deg_body, None)
    plsc.subcore_barrier()

    @pl.when(c == 0)
    def _():
        pltpu.sync_copy(deg_sh.at[pl.ds(s * spt, spt)], degv)

        def nwt(i, _):
            degv[pl.ds(16 * i, 16)] = _rsqrt16(degv[pl.ds(16 * i, 16)])
            return _
        lax.fori_loop(0, spt // 16, nwt, None)
        pltpu.sync_copy(degv, dinv_o.at[pl.ds(s * spt, spt)])

    g0 = c * NHALF + s * TR

    def x_body(ch, _):
        gb = g0 + ch * CB
        eb = jnp.minimum(gb, N - CB)
        pltpu.sync_copy(emb_hbm.at[pl.ds(eb, CB)], embv)
        pltpu.sync_copy(deg_sh.at[pl.ds(gb, CB)], dv)
        for k in range(CB // 16):
            dv[pl.ds(16 * k, 16)] = _rsqrt16(dv[pl.ds(16 * k, 16)])
        w0 = wv[pl.ds(0, 16)][0]

        def row_grp(g, _):
            dvec = dv[pl.ds(16 * g, 16)]
            for k in range(16):
                d = dvec[k]
                r = 16 * g + k
                for v in range(D // 16):
                    e = embv[r, pl.ds(16 * v, 16)]
                    xtv[r, pl.ds(16 * v, 16)] = d * e
                    ov[r, pl.ds(16 * v, 16)] = w0 * e
            return _
        lax.fori_loop(0, CB // 16, row_grp, None)
        pltpu.sync_copy(xtv, xt0_o.at[pl.ds(gb, CB)])
        pltpu.sync_copy(ov, out0_o.at[pl.ds(gb, CB)])
        return _
    lax.fori_loop(0, TR // CB, x_body, None)


def _make_layer(widx: int, last: bool):
    outs = [jax.ShapeDtypeStruct((NPAD, D), _f32)]
    if not last:
        outs.append(jax.ShapeDtypeStruct((NPAD, D), _f32))

    @functools.partial(
        pl.kernel,
        out_type=tuple(outs),
        mesh=_MESH,
        compiler_params=_CP,
        scratch_types=(
            pltpu.VMEM_SHARED((NHALF, D), _f32),
            pltpu.VMEM((EB,), _i32),
            pltpu.VMEM((EB,), _i32),
            pltpu.VMEM((ECB, D), _f32),
            pltpu.VMEM((ECB, D), _f32),
            pltpu.VMEM((CB, D), _f32),
            pltpu.VMEM((CB,), _f32),
            pltpu.VMEM((16,), _f32),
        ),
    )
    def _layer(row_hbm, col_hbm, xt_in, out_in, dinv_hbm, w_hbm,
               out_o, *rest):
        if last:
            (acc, rowB, clB, msg, msg2, xtv, dv, wv) = rest
            xt_o = None
        else:
            (xt_o, acc, rowB, clB, msg, msg2, xtv, dv, wv) = rest
        c = lax.axis_index("c")
        s = lax.axis_index("s")

        _zero_rows(msg, CB)

        def z_body(i, _):
            pltpu.sync_copy(msg.at[pl.ds(0, CB)],
                            acc.at[pl.ds(s * TR + i * CB, CB)])
            return _
        lax.fori_loop(0, TR // CB, z_body, None)
        pltpu.sync_copy(w_hbm, wv)

        plsc.subcore_barrier()

        base = c * NHALF

        def blk_body(b, _):
            e0 = s * EPT + b * EB
            pltpu.sync_copy(row_hbm.at[pl.ds(e0, EB)], rowB)
            pltpu.sync_copy(col_hbm.at[pl.ds(e0, EB)], clB)

            def mk(i, _):
                lc = clB[pl.ds(16 * i, 16)] - base
                m = (lc >= 0) & (lc < NHALF)
                clB[pl.ds(16 * i, 16)] = jnp.where(m, lc, -1)
                return _
            lax.fori_loop(0, EB // 16, mk, None)

            def ch(j, _):
                pltpu.sync_copy(xt_in.at[rowB.at[pl.ds(j * ECB, ECB)]], msg)
                pltpu.sync_copy(
                    msg,
                    acc.at[plsc.Indices(clB.at[pl.ds(j * ECB, ECB)],
                                        ignored_value=-1)],
                    add=True,
                )
                return _
            lax.fori_loop(0, NCB, ch, None)
            return _
        lax.fori_loop(0, NEB, blk_body, None)
        plsc.subcore_barrier()

        g0 = c * NHALF + s * TR
        w = wv[pl.ds(0, 16)][widx]

        def wb_body(ch, _):
            gb = g0 + ch * CB
            lb = s * TR + ch * CB
            pltpu.sync_copy(acc.at[pl.ds(lb, CB)], msg.at[pl.ds(0, CB)])
            pltpu.sync_copy(dinv_hbm.at[pl.ds(gb, CB)], dv)
            pltpu.sync_copy(out_in.at[pl.ds(gb, CB)], msg2.at[pl.ds(0, CB)])

            def row_grp(g, _):
                dvec = dv[pl.ds(16 * g, 16)]
                for k in range(16):
                    d = dvec[k]
                    r = 16 * g + k
                    for v in range(D // 16):
                        a = msg[r, pl.ds(16 * v, 16)]
                        xn = d * a
                        msg2[r, pl.ds(16 * v, 16)] = (
                            msg2[r, pl.ds(16 * v, 16)] + w * xn
                        )
                        if not last:
                            xtv[r, pl.ds(16 * v, 16)] = d * xn
                return _
            lax.fori_loop(0, CB // 16, row_grp, None)
            pltpu.sync_copy(msg2.at[pl.ds(0, CB)], out_o.at[pl.ds(gb, CB)])
            if not last:
                pltpu.sync_copy(xtv, xt_o.at[pl.ds(gb, CB)])
            return _
        lax.fori_loop(0, TR // CB, wb_body, None)

    return _layer


@functools.partial(
    pl.kernel,
    out_type=jax.ShapeDtypeStruct((NLP,), _f32),
    mesh=_MESH,
    compiler_params=_CP,
    scratch_types=(
        pltpu.VMEM((LCB,), _i32),
        pltpu.VMEM((LCB,), _i32),
        pltpu.VMEM((LCB, D), _f32),
        pltpu.VMEM((LCB, D), _f32),
        pltpu.VMEM((LCB,), _f32),
    ),
)
def _k4(out_hbm, lsrc, ldst, res_o, siv, div_, av, bv, rv):
    c = lax.axis_index("c")
    s = lax.axis_index("s")
    wid = s * NC + c
    base0 = wid * LPT

    def ch_body(ch, _):
        eb = base0 + ch * LCB
        pltpu.sync_copy(lsrc.at[pl.ds(eb, LCB)], siv)
        pltpu.sync_copy(ldst.at[pl.ds(eb, LCB)], div_)
        pltpu.sync_copy(out_hbm.at[siv], av)
        pltpu.sync_copy(out_hbm.at[div_], bv)

        lane = lax.iota(_i32, 16)

        def row_grp(g, _):
            t = jnp.zeros((16,), _f32)
            for k in range(16):
                r = 16 * g + k
                acc = av[r, pl.ds(0, 16)] * bv[r, pl.ds(0, 16)]
                for v in range(1, D // 16):
                    acc = acc + (av[r, pl.ds(16 * v, 16)]
                                 * bv[r, pl.ds(16 * v, 16)])
                t = jnp.where(lane == k, jnp.sum(acc), t)
            rv[pl.ds(16 * g, 16)] = t
            return _
        lax.fori_loop(0, LCB // 16, row_grp, None)
        pltpu.sync_copy(rv, res_o.at[pl.ds(eb, LCB)])
        return _
    lax.fori_loop(0, LNCH, ch_body, None)


_LAYER1 = _make_layer(1, last=False)
_LAYER2 = _make_layer(2, last=False)
_LAYER3 = _make_layer(3, last=True)


def kernel(edge_index, edge_label_index, embedding, alpha):
    row = edge_index[0]
    col = edge_index[1]
    w = jax.nn.softmax(alpha, axis=-1)
    w16 = jnp.zeros((16,), _f32).at[:4].set(w)
    lsrc = jnp.zeros((NLP,), _i32).at[:NL].set(edge_label_index[0])
    ldst = jnp.zeros((NLP,), _i32).at[:NL].set(edge_label_index[1])

    dinv, xt0, out0 = _k0(col, embedding, w16)
    out1, xt1 = _LAYER1(row, col, xt0, out0, dinv, w16)
    out2, xt2 = _LAYER2(row, col, xt1, out1, dinv, w16)
    (out3,) = _LAYER3(row, col, xt2, out2, dinv, w16)
    res = _k4(out3, lsrc, ldst)
    return res[:NL]
